# Initial kernel scaffold; baseline (speedup 1.0000x reference)
import jax
import jax.numpy as jnp
from jax.experimental import pallas as pl


def kernel(embeddings, instance_ids):
    raise NotImplementedError("write your pallas kernel here")


# speedup vs baseline: 10.9920x; 10.9920x over previous
import jax
import jax.numpy as jnp
from jax import lax
from jax.experimental import pallas as pl
from jax.experimental.pallas import tpu as pltpu
from jax.experimental.pallas import tpu_sc as plsc

_DELTA_V = 0.5
_DELTA_D = 1.5
_ALPHA = 1.0
_BETA = 1.0
_GAMMA = 0.001
_K = 64

_B = 8
_N = 32768
_D = 16
_L = 16

_WPR = 4
_PTS = _N // _WPR
_CH = 2048
_NCH = _PTS // _CH
_TPC = _CH // _L

_ACC = 1024 + 64 + 16


def _rsqrt_nr(x):
    i = plsc.bitcast(x, jnp.int32)
    i = jnp.int32(0x5F3759DF) - lax.shift_right_logical(i, 1)
    y = plsc.bitcast(i, jnp.float32)
    xh = x * 0.5
    for _ in range(3):
        y = y * (1.5 - xh * y * y)
    return y


def _sqrt16(x):
    return x * _rsqrt_nr(jnp.maximum(x, 1e-30))


def _body(emb_hbm, ids_hbm, out_hbm, xbuf, idsbuf, acc, tmp, mm, mt, pi,
          res, sh_acc, sh_mm, sh_pi):
    c = lax.axis_index("c")
    s = lax.axis_index("s")
    row = c * 4 + s // _WPR
    q = s % _WPR
    qbase = q * _PTS
    slot = s // _WPR

    iota = lax.iota(jnp.int32, _L)
    iotad = lax.shift_left(iota, 4)
    ones = jnp.ones((_L,), jnp.float32)

    def _z(i, _):
        acc[pl.ds(i * _L, _L)] = jnp.zeros((_L,), jnp.float32)
        return 0

    lax.fori_loop(0, _ACC // _L, _z, 0)
    for kv in range(4):
        pi[pl.ds(kv * _L, _L)] = jnp.zeros((_L,), jnp.float32)

    def _chunk1(ci, _):
        start = qbase + ci * _CH
        pltpu.sync_copy(emb_hbm.at[row, pl.ds(start * _D, _CH * _D)], xbuf)
        pltpu.sync_copy(ids_hbm.at[row, pl.ds(start, _CH)], idsbuf)

        def _tile(t, _):
            ids16 = idsbuf[pl.ds(t * _L, _L)]
            idsx = lax.shift_left(ids16, 4)
            plsc.addupdate_scatter(acc, [ids16 + 1024], ones)
            pix = jnp.full((_L,), t * _L * _D, jnp.int32) + iotad
            for d in range(_D):
                col = plsc.load_gather(xbuf, [pix + d])
                plsc.addupdate_scatter(acc, [idsx + d], col)
            return 0

        lax.fori_loop(0, _TPC, _tile, 0)
        return 0

    lax.fori_loop(0, _NCH, _chunk1, 0)

    pltpu.sync_copy(acc, sh_acc.at[pl.ds(s * _ACC, _ACC)])
    plsc.subcore_barrier()

    @pl.when(q == 0)
    def _reduce():
        for part in range(1, _WPR):
            pltpu.sync_copy(sh_acc.at[pl.ds((s + part) * _ACC, _ACC)], tmp)

            def _add(i, _):
                acc[pl.ds(i * _L, _L)] = acc[pl.ds(i * _L, _L)] + tmp[
                    pl.ds(i * _L, _L)
                ]
                return 0

            lax.fori_loop(0, _ACC // _L, _add, 0)

        for kv in range(4):
            cnt16 = acc[pl.ds(1024 + kv * _L, _L)]
            acc[pl.ds(1024 + kv * _L, _L)] = 1.0 / jnp.maximum(cnt16, 1.0)

        def _mean(k, _):
            inv = acc[pl.ds(1024 + k, _L)][0]
            mm[pl.ds(k * _L, _L)] = acc[pl.ds(k * _L, _L)] * inv
            return 0

        lax.fori_loop(0, _K, _mean, 0)
        pltpu.sync_copy(mm, sh_mm.at[pl.ds(slot * 1024, 1024)])

    plsc.subcore_barrier()
    pltpu.sync_copy(sh_mm.at[pl.ds(slot * 1024, 1024)], mm)

    def _chunk2(ci, _):
        start = qbase + ci * _CH
        pltpu.sync_copy(emb_hbm.at[row, pl.ds(start * _D, _CH * _D)], xbuf)
        pltpu.sync_copy(ids_hbm.at[row, pl.ds(start, _CH)], idsbuf)

        def _tile(t, _):
            ids16 = idsbuf[pl.ds(t * _L, _L)]
            idsx = lax.shift_left(ids16, 4)
            pix = jnp.full((_L,), t * _L * _D, jnp.int32) + iotad
            ssq = jnp.full((_L,), 1e-12, jnp.float32)
            for d in range(_D):
                col = plsc.load_gather(xbuf, [pix + d])
                mcol = plsc.load_gather(mm, [idsx + d])
                diff = col - mcol
                ssq = ssq + diff * diff
            dist = ssq * _rsqrt_nr(ssq)
            hg = jnp.maximum(dist - _DELTA_V, 0.0)
            plsc.addupdate_scatter(pi, [ids16], hg * hg)
            return 0

        lax.fori_loop(0, _TPC, _tile, 0)
        return 0

    lax.fori_loop(0, _NCH, _chunk2, 0)

    pltpu.sync_copy(pi, sh_pi.at[pl.ds(s * 64, 64)])
    plsc.subcore_barrier()

    @pl.when(q == 0)
    def _finish():
        for part in range(1, _WPR):
            pltpu.sync_copy(sh_pi.at[pl.ds((s + part) * 64, 64)], tmp.at[pl.ds(0, 64)])
            for kv in range(4):
                pi[pl.ds(kv * _L, _L)] = pi[pl.ds(kv * _L, _L)] + tmp[
                    pl.ds(kv * _L, _L)
                ]

        var = jnp.float32(0.0)
        for kv in range(4):
            inv16 = acc[pl.ds(1024 + kv * _L, _L)]
            t16 = pi[pl.ds(kv * _L, _L)] * inv16
            var = var + jnp.sum(t16)
        var = var * jnp.float32(1.0 / _K)

        for d in range(_D):
            for jv in range(4):
                jidx = lax.shift_left(
                    jnp.full((_L,), jv * _L, jnp.int32) + iota, 4
                )
                mt[pl.ds(d * _K + jv * _L, _L)] = plsc.load_gather(
                    mm, [jidx + d]
                )

        dloss = jnp.float32(0.0)
        for jv in range(4):
            mtv = [mt[pl.ds(d * _K + jv * _L, _L)] for d in range(_D)]
            jids = jnp.full((_L,), jv * _L, jnp.int32) + iota

            def _prow(i, dl):
                mi = mm[pl.ds(i * _L, _L)]
                sq = jnp.zeros((_L,), jnp.float32)
                for d in range(_D):
                    diff = mtv[d] - mi[d]
                    sq = sq + diff * diff
                pd = _sqrt16(sq)
                h = jnp.maximum(2.0 * _DELTA_D - pd, 0.0)
                h = jnp.where(jids == i, 0.0, h * h)
                return dl + jnp.sum(h)

            dloss = lax.fori_loop(0, _K, _prow, dloss)
        dloss = dloss * jnp.float32(0.5 / (_K * (_K - 1) / 2.0))

        reg = jnp.float32(0.0)
        for jv in range(4):
            sq = jnp.full((_L,), 1e-12, jnp.float32)
            for d in range(_D):
                v = mt[pl.ds(d * _K + jv * _L, _L)]
                sq = sq + v * v
            reg = reg + jnp.sum(_sqrt16(sq))
        reg = reg * jnp.float32(1.0 / _K)

        out16 = jnp.where(
            iota == 0,
            var,
            jnp.where(iota == 1, dloss, jnp.where(iota == 2, reg, 0.0)),
        )
        res[...] = out16
        pltpu.sync_copy(res, out_hbm.at[row])


def _make_kernel():
    mesh = plsc.VectorSubcoreMesh(core_axis_name="c", subcore_axis_name="s")
    scratch = [
        pltpu.VMEM((_CH * _D,), jnp.float32),
        pltpu.VMEM((_CH,), jnp.int32),
        pltpu.VMEM((_ACC,), jnp.float32),
        pltpu.VMEM((_ACC,), jnp.float32),
        pltpu.VMEM((1024,), jnp.float32),
        pltpu.VMEM((1024,), jnp.float32),
        pltpu.VMEM((64,), jnp.float32),
        pltpu.VMEM((_L,), jnp.float32),
        pltpu.VMEM_SHARED((16 * _ACC,), jnp.float32),
        pltpu.VMEM_SHARED((4 * 1024,), jnp.float32),
        pltpu.VMEM_SHARED((16 * 64,), jnp.float32),
    ]
    return pl.kernel(
        _body,
        mesh=mesh,
        out_type=jax.ShapeDtypeStruct((_B, _L), jnp.float32),
        scratch_types=scratch,
        compiler_params=pltpu.CompilerParams(needs_layout_passes=False),
    )


@jax.jit
def kernel(embeddings, instance_ids):
    out = _make_kernel()(embeddings.reshape(_B, _N * _D), instance_ids)
    var = jnp.mean(out[:, 0])
    dist = jnp.mean(out[:, 1])
    reg = jnp.mean(out[:, 2])
    total = _ALPHA * var + _BETA * dist + _GAMMA * reg
    return (total, var, dist, reg)


# speedup vs baseline: 11.7534x; 1.0693x over previous
import jax
import jax.numpy as jnp
from jax import lax
from jax.experimental import pallas as pl
from jax.experimental.pallas import tpu as pltpu
from jax.experimental.pallas import tpu_sc as plsc

_DELTA_V = 0.5
_DELTA_D = 1.5
_ALPHA = 1.0
_BETA = 1.0
_GAMMA = 0.001
_K = 64

_B = 8
_N = 32768
_D = 16
_L = 16

_WPR = 4
_PTS = _N // _WPR
_CH = 2048
_NCH = _PTS // _CH
_TPC = _CH // _L
_U = 4

_ACC = 1024 + 64 + 16


def _rsqrt_nr(x):
    i = plsc.bitcast(x, jnp.int32)
    i = jnp.int32(0x5F3759DF) - lax.shift_right_logical(i, 1)
    y = plsc.bitcast(i, jnp.float32)
    xh = x * 0.5
    for _ in range(3):
        y = y * (1.5 - xh * y * y)
    return y


def _sqrt16(x):
    return x * _rsqrt_nr(jnp.maximum(x, 1e-30))


def _body(emb_hbm, ids_hbm, out_hbm, xb0, xb1, idsbuf, acc, tmp, mm, mt, pi,
          res, sem0, sem1, sh_acc, sh_mm, sh_pi):
    c = lax.axis_index("c")
    s = lax.axis_index("s")
    row = c * 4 + s // _WPR
    q = s % _WPR
    qbase = q * _PTS
    slot = s // _WPR

    iota = lax.iota(jnp.int32, _L)
    iotad = lax.shift_left(iota, 4)
    ones = jnp.ones((_L,), jnp.float32)

    def _z(i, _):
        acc[pl.ds(i * _L, _L)] = jnp.zeros((_L,), jnp.float32)
        return 0

    lax.fori_loop(0, _ACC // _L, _z, 0)
    for kv in range(4):
        pi[pl.ds(kv * _L, _L)] = jnp.zeros((_L,), jnp.float32)

    pltpu.sync_copy(ids_hbm.at[row, pl.ds(qbase, _PTS)], idsbuf)

    def _emb_copy(ci, xb, sem):
        start = qbase + ci * _CH
        return pltpu.make_async_copy(
            emb_hbm.at[row, pl.ds(start * _D, _CH * _D)], xb, sem
        )

    def _buf(ci):
        return (xb0, sem0) if ci % 2 == 0 else (xb1, sem1)

    _emb_copy(0, xb0, sem0).start()
    for ci in range(_NCH):
        xb, sem = _buf(ci)
        if ci + 1 < _NCH:
            nxb, nsem = _buf(ci + 1)
            _emb_copy(ci + 1, nxb, nsem).start()
        _emb_copy(ci, xb, sem).wait()

        def _tile4(t4, _, xb=xb, ci=ci):
            for u in range(_U):
                idbase = ci * _CH + t4 * (_U * _L) + u * _L
                ids16 = idsbuf[pl.ds(idbase, _L)]
                idsx = lax.shift_left(ids16, 4)
                plsc.addupdate_scatter(acc, [ids16 + 1024], ones)
                pix = iotad + (t4 * (_U * _L * _D) + u * _L * _D)
                for d in range(_D):
                    col = plsc.load_gather(xb, [pix + d])
                    plsc.addupdate_scatter(acc, [idsx + d], col)
            return 0

        lax.fori_loop(0, _TPC // _U, _tile4, 0)

    pltpu.sync_copy(acc, sh_acc.at[pl.ds(s * _ACC, _ACC)])
    plsc.subcore_barrier()

    @pl.when(q == 0)
    def _reduce():
        for part in range(1, _WPR):
            pltpu.sync_copy(sh_acc.at[pl.ds((s + part) * _ACC, _ACC)], tmp)

            def _add(i, _):
                acc[pl.ds(i * _L, _L)] = acc[pl.ds(i * _L, _L)] + tmp[
                    pl.ds(i * _L, _L)
                ]
                return 0

            lax.fori_loop(0, _ACC // _L, _add, 0)

        for kv in range(4):
            cnt16 = acc[pl.ds(1024 + kv * _L, _L)]
            acc[pl.ds(1024 + kv * _L, _L)] = 1.0 / jnp.maximum(cnt16, 1.0)

        def _mean(k, _):
            inv = acc[pl.ds(1024 + k, _L)][0]
            mm[pl.ds(k * _L, _L)] = acc[pl.ds(k * _L, _L)] * inv
            return 0

        lax.fori_loop(0, _K, _mean, 0)
        pltpu.sync_copy(mm, sh_mm.at[pl.ds(slot * 1024, 1024)])

    plsc.subcore_barrier()
    pltpu.sync_copy(sh_mm.at[pl.ds(slot * 1024, 1024)], mm)

    _emb_copy(0, xb0, sem0).start()
    for ci in range(_NCH):
        xb, sem = _buf(ci)
        if ci + 1 < _NCH:
            nxb, nsem = _buf(ci + 1)
            _emb_copy(ci + 1, nxb, nsem).start()
        _emb_copy(ci, xb, sem).wait()

        def _tile4(t4, _, xb=xb, ci=ci):
            for u in range(_U):
                idbase = ci * _CH + t4 * (_U * _L) + u * _L
                ids16 = idsbuf[pl.ds(idbase, _L)]
                idsx = lax.shift_left(ids16, 4)
                pix = iotad + (t4 * (_U * _L * _D) + u * _L * _D)
                ssq = jnp.full((_L,), 1e-12, jnp.float32)
                for d in range(_D):
                    col = plsc.load_gather(xb, [pix + d])
                    mcol = plsc.load_gather(mm, [idsx + d])
                    diff = col - mcol
                    ssq = ssq + diff * diff
                dist = ssq * _rsqrt_nr(ssq)
                hg = jnp.maximum(dist - _DELTA_V, 0.0)
                plsc.addupdate_scatter(pi, [ids16], hg * hg)
            return 0

        lax.fori_loop(0, _TPC // _U, _tile4, 0)

    pltpu.sync_copy(pi, sh_pi.at[pl.ds(s * 64, 64)])
    plsc.subcore_barrier()

    @pl.when(q == 0)
    def _finish():
        for part in range(1, _WPR):
            pltpu.sync_copy(sh_pi.at[pl.ds((s + part) * 64, 64)], tmp.at[pl.ds(0, 64)])
            for kv in range(4):
                pi[pl.ds(kv * _L, _L)] = pi[pl.ds(kv * _L, _L)] + tmp[
                    pl.ds(kv * _L, _L)
                ]

        var = jnp.float32(0.0)
        for kv in range(4):
            inv16 = acc[pl.ds(1024 + kv * _L, _L)]
            t16 = pi[pl.ds(kv * _L, _L)] * inv16
            var = var + jnp.sum(t16)
        var = var * jnp.float32(1.0 / _K)

        for d in range(_D):
            for jv in range(4):
                jidx = lax.shift_left(
                    jnp.full((_L,), jv * _L, jnp.int32) + iota, 4
                )
                mt[pl.ds(d * _K + jv * _L, _L)] = plsc.load_gather(
                    mm, [jidx + d]
                )

        dloss = jnp.float32(0.0)
        for jv in range(4):
            mtv = [mt[pl.ds(d * _K + jv * _L, _L)] for d in range(_D)]
            jids = jnp.full((_L,), jv * _L, jnp.int32) + iota

            def _prow(i, dl):
                mi = mm[pl.ds(i * _L, _L)]
                sq = jnp.zeros((_L,), jnp.float32)
                for d in range(_D):
                    diff = mtv[d] - mi[d]
                    sq = sq + diff * diff
                pd = _sqrt16(sq)
                h = jnp.maximum(2.0 * _DELTA_D - pd, 0.0)
                h = jnp.where(jids == i, 0.0, h * h)
                return dl + jnp.sum(h)

            dloss = lax.fori_loop(0, _K, _prow, dloss)
        dloss = dloss * jnp.float32(0.5 / (_K * (_K - 1) / 2.0))

        reg = jnp.float32(0.0)
        for jv in range(4):
            sq = jnp.full((_L,), 1e-12, jnp.float32)
            for d in range(_D):
                v = mt[pl.ds(d * _K + jv * _L, _L)]
                sq = sq + v * v
            reg = reg + jnp.sum(_sqrt16(sq))
        reg = reg * jnp.float32(1.0 / _K)

        out16 = jnp.where(
            iota == 0,
            var,
            jnp.where(iota == 1, dloss, jnp.where(iota == 2, reg, 0.0)),
        )
        res[...] = out16
        pltpu.sync_copy(res, out_hbm.at[row])


def _make_kernel():
    mesh = plsc.VectorSubcoreMesh(core_axis_name="c", subcore_axis_name="s")
    scratch = [
        pltpu.VMEM((_CH * _D,), jnp.float32),
        pltpu.VMEM((_CH * _D,), jnp.float32),
        pltpu.VMEM((_PTS,), jnp.int32),
        pltpu.VMEM((_ACC,), jnp.float32),
        pltpu.VMEM((_ACC,), jnp.float32),
        pltpu.VMEM((1024,), jnp.float32),
        pltpu.VMEM((1024,), jnp.float32),
        pltpu.VMEM((64,), jnp.float32),
        pltpu.VMEM((_L,), jnp.float32),
        pltpu.SemaphoreType.DMA,
        pltpu.SemaphoreType.DMA,
        pltpu.VMEM_SHARED((16 * _ACC,), jnp.float32),
        pltpu.VMEM_SHARED((4 * 1024,), jnp.float32),
        pltpu.VMEM_SHARED((16 * 64,), jnp.float32),
    ]
    return pl.kernel(
        _body,
        mesh=mesh,
        out_type=jax.ShapeDtypeStruct((_B, _L), jnp.float32),
        scratch_types=scratch,
        compiler_params=pltpu.CompilerParams(
            needs_layout_passes=False, use_tc_tiling_on_sc=False
        ),
    )


@jax.jit
def kernel(embeddings, instance_ids):
    out = _make_kernel()(embeddings.reshape(_B, _N * _D), instance_ids)
    var = jnp.mean(out[:, 0])
    dist = jnp.mean(out[:, 1])
    reg = jnp.mean(out[:, 2])
    total = _ALPHA * var + _BETA * dist + _GAMMA * reg
    return (total, var, dist, reg)


# speedup vs baseline: 19.5242x; 1.6611x over previous
import jax
import jax.numpy as jnp
from jax import lax
from jax.experimental import pallas as pl
from jax.experimental.pallas import tpu as pltpu
from jax.experimental.pallas import tpu_sc as plsc

_DELTA_V = 0.5
_DELTA_D = 1.5
_ALPHA = 1.0
_BETA = 1.0
_GAMMA = 0.001
_K = 64

_B = 8
_N = 32768
_D = 16
_L = 16

_WPR = 4
_PTS = _N // _WPR
_CH = 2048
_NCH = _PTS // _CH
_TPC = _CH // _L
_U = 4

_ACC = 1024 + 64 + 16


def _rsqrt_nr(x):
    i = plsc.bitcast(x, jnp.int32)
    i = jnp.int32(0x5F3759DF) - lax.shift_right_logical(i, 1)
    y = plsc.bitcast(i, jnp.float32)
    xh = x * 0.5
    for _ in range(3):
        y = y * (1.5 - xh * y * y)
    return y


def _sqrt16(x):
    return x * _rsqrt_nr(jnp.maximum(x, 1e-30))


def _body(emb_hbm, ids_hbm, out_hbm, xb0, xb1, idsbuf, acc, tmp, mm, mt, pi,
          res, sem0, sem1, sh_acc, sh_mm, sh_pi):
    c = lax.axis_index("c")
    s = lax.axis_index("s")
    row = c * 4 + s // _WPR
    q = s % _WPR
    qbase = q * _PTS
    slot = s // _WPR

    iota = lax.iota(jnp.int32, _L)
    iotad = lax.shift_left(iota, 4)
    ones = jnp.ones((_L,), jnp.float32)

    def _z(i, _):
        acc[pl.ds(i * _L, _L)] = jnp.zeros((_L,), jnp.float32)
        return 0

    lax.fori_loop(0, _ACC // _L, _z, 0)
    for kv in range(4):
        pi[pl.ds(kv * _L, _L)] = jnp.zeros((_L,), jnp.float32)

    pltpu.sync_copy(ids_hbm.at[row, pl.ds(qbase, _PTS)], idsbuf)

    def _emb_copy(ci, xb, sem):
        start = qbase + ci * _CH
        return pltpu.make_async_copy(
            emb_hbm.at[row, pl.ds(start * _D, _CH * _D)], xb, sem
        )

    def _buf(ci):
        return (xb0, sem0) if ci % 2 == 0 else (xb1, sem1)

    _emb_copy(0, xb0, sem0).start()
    for ci in range(_NCH):
        xb, sem = _buf(ci)
        if ci + 1 < _NCH:
            nxb, nsem = _buf(ci + 1)
            _emb_copy(ci + 1, nxb, nsem).start()
        _emb_copy(ci, xb, sem).wait()

        def _tile4(t4, _, xb=xb, ci=ci):
            for u in range(_U):
                idbase = ci * _CH + t4 * (_U * _L) + u * _L
                ids16 = idsbuf[pl.ds(idbase, _L)]
                plsc.addupdate_scatter(acc, [ids16 + 1024], ones)
                off = t4 * (_U * _L * _D) + u * _L * _D
                i0 = ids16[0]

                @pl.when(i0 == ids16[_L - 1])
                def _fast(off=off, i0=i0):
                    vs = [xb[pl.ds(off + p * _D, _D)] for p in range(_L)]
                    while len(vs) > 1:
                        vs = [
                            vs[i] + vs[i + 1] for i in range(0, len(vs), 2)
                        ]
                    plsc.addupdate(acc.at[pl.ds(i0 * _D, _D)], vs[0])

                @pl.when(i0 != ids16[_L - 1])
                def _slow(off=off, ids16=ids16):
                    idsx = lax.shift_left(ids16, 4)
                    pix = iotad + off
                    for d in range(_D):
                        col = plsc.load_gather(xb, [pix + d])
                        plsc.addupdate_scatter(acc, [idsx + d], col)

            return 0

        lax.fori_loop(0, _TPC // _U, _tile4, 0)

    pltpu.sync_copy(acc, sh_acc.at[pl.ds(s * _ACC, _ACC)])
    plsc.subcore_barrier()

    @pl.when(q == 0)
    def _reduce():
        for part in range(1, _WPR):
            pltpu.sync_copy(sh_acc.at[pl.ds((s + part) * _ACC, _ACC)], tmp)

            def _add(i, _):
                acc[pl.ds(i * _L, _L)] = acc[pl.ds(i * _L, _L)] + tmp[
                    pl.ds(i * _L, _L)
                ]
                return 0

            lax.fori_loop(0, _ACC // _L, _add, 0)

        for kv in range(4):
            cnt16 = acc[pl.ds(1024 + kv * _L, _L)]
            acc[pl.ds(1024 + kv * _L, _L)] = 1.0 / jnp.maximum(cnt16, 1.0)

        def _mean(k, _):
            inv = acc[pl.ds(1024 + k, _L)][0]
            mm[pl.ds(k * _L, _L)] = acc[pl.ds(k * _L, _L)] * inv
            return 0

        lax.fori_loop(0, _K, _mean, 0)
        pltpu.sync_copy(mm, sh_mm.at[pl.ds(slot * 1024, 1024)])

    plsc.subcore_barrier()
    pltpu.sync_copy(sh_mm.at[pl.ds(slot * 1024, 1024)], mm)

    _emb_copy(0, xb0, sem0).start()
    for ci in range(_NCH):
        xb, sem = _buf(ci)
        if ci + 1 < _NCH:
            nxb, nsem = _buf(ci + 1)
            _emb_copy(ci + 1, nxb, nsem).start()
        _emb_copy(ci, xb, sem).wait()

        def _tile4(t4, _, xb=xb, ci=ci):
            for u in range(_U):
                idbase = ci * _CH + t4 * (_U * _L) + u * _L
                ids16 = idsbuf[pl.ds(idbase, _L)]
                pix = iotad + (t4 * (_U * _L * _D) + u * _L * _D)
                i0 = ids16[0]

                def _ssq_from(mcols, pix=pix, xb=xb):
                    ssq = jnp.full((_L,), 1e-12, jnp.float32)
                    for d in range(_D):
                        col = plsc.load_gather(xb, [pix + d])
                        diff = col - mcols[d]
                        ssq = ssq + diff * diff
                    return ssq

                def _hinge(ssq, ids16=ids16):
                    dist = ssq * _rsqrt_nr(ssq)
                    hg = jnp.maximum(dist - _DELTA_V, 0.0)
                    plsc.addupdate_scatter(pi, [ids16], hg * hg)

                @pl.when(i0 == ids16[_L - 1])
                def _fast(i0=i0, _ssq_from=_ssq_from, _hinge=_hinge):
                    mv = mm[pl.ds(i0 * _D, _D)]
                    _hinge(_ssq_from([mv[d] for d in range(_D)]))

                @pl.when(i0 != ids16[_L - 1])
                def _slow(ids16=ids16, _ssq_from=_ssq_from, _hinge=_hinge):
                    idsx = lax.shift_left(ids16, 4)
                    mcols = [
                        plsc.load_gather(mm, [idsx + d]) for d in range(_D)
                    ]
                    _hinge(_ssq_from(mcols))

            return 0

        lax.fori_loop(0, _TPC // _U, _tile4, 0)

    pltpu.sync_copy(pi, sh_pi.at[pl.ds(s * 64, 64)])
    plsc.subcore_barrier()

    @pl.when(q == 0)
    def _finish():
        for part in range(1, _WPR):
            pltpu.sync_copy(sh_pi.at[pl.ds((s + part) * 64, 64)], tmp.at[pl.ds(0, 64)])
            for kv in range(4):
                pi[pl.ds(kv * _L, _L)] = pi[pl.ds(kv * _L, _L)] + tmp[
                    pl.ds(kv * _L, _L)
                ]

        var = jnp.float32(0.0)
        for kv in range(4):
            inv16 = acc[pl.ds(1024 + kv * _L, _L)]
            t16 = pi[pl.ds(kv * _L, _L)] * inv16
            var = var + jnp.sum(t16)
        var = var * jnp.float32(1.0 / _K)

        for d in range(_D):
            for jv in range(4):
                jidx = lax.shift_left(
                    jnp.full((_L,), jv * _L, jnp.int32) + iota, 4
                )
                mt[pl.ds(d * _K + jv * _L, _L)] = plsc.load_gather(
                    mm, [jidx + d]
                )

        dloss = jnp.float32(0.0)
        for jv in range(4):
            mtv = [mt[pl.ds(d * _K + jv * _L, _L)] for d in range(_D)]
            jids = jnp.full((_L,), jv * _L, jnp.int32) + iota

            def _prow(i, dl):
                mi = mm[pl.ds(i * _L, _L)]
                sq = jnp.zeros((_L,), jnp.float32)
                for d in range(_D):
                    diff = mtv[d] - mi[d]
                    sq = sq + diff * diff
                pd = _sqrt16(sq)
                h = jnp.maximum(2.0 * _DELTA_D - pd, 0.0)
                h = jnp.where(jids == i, 0.0, h * h)
                return dl + jnp.sum(h)

            dloss = lax.fori_loop(0, _K, _prow, dloss)
        dloss = dloss * jnp.float32(0.5 / (_K * (_K - 1) / 2.0))

        reg = jnp.float32(0.0)
        for jv in range(4):
            sq = jnp.full((_L,), 1e-12, jnp.float32)
            for d in range(_D):
                v = mt[pl.ds(d * _K + jv * _L, _L)]
                sq = sq + v * v
            reg = reg + jnp.sum(_sqrt16(sq))
        reg = reg * jnp.float32(1.0 / _K)

        out16 = jnp.where(
            iota == 0,
            var,
            jnp.where(iota == 1, dloss, jnp.where(iota == 2, reg, 0.0)),
        )
        res[...] = out16
        pltpu.sync_copy(res, out_hbm.at[row])


def _make_kernel():
    mesh = plsc.VectorSubcoreMesh(core_axis_name="c", subcore_axis_name="s")
    scratch = [
        pltpu.VMEM((_CH * _D,), jnp.float32),
        pltpu.VMEM((_CH * _D,), jnp.float32),
        pltpu.VMEM((_PTS,), jnp.int32),
        pltpu.VMEM((_ACC,), jnp.float32),
        pltpu.VMEM((_ACC,), jnp.float32),
        pltpu.VMEM((1024,), jnp.float32),
        pltpu.VMEM((1024,), jnp.float32),
        pltpu.VMEM((64,), jnp.float32),
        pltpu.VMEM((_L,), jnp.float32),
        pltpu.SemaphoreType.DMA,
        pltpu.SemaphoreType.DMA,
        pltpu.VMEM_SHARED((16 * _ACC,), jnp.float32),
        pltpu.VMEM_SHARED((4 * 1024,), jnp.float32),
        pltpu.VMEM_SHARED((16 * 64,), jnp.float32),
    ]
    return pl.kernel(
        _body,
        mesh=mesh,
        out_type=jax.ShapeDtypeStruct((_B, _L), jnp.float32),
        scratch_types=scratch,
        compiler_params=pltpu.CompilerParams(
            needs_layout_passes=False, use_tc_tiling_on_sc=False
        ),
    )


@jax.jit
def kernel(embeddings, instance_ids):
    out = _make_kernel()(embeddings.reshape(_B, _N * _D), instance_ids)
    var = jnp.mean(out[:, 0])
    dist = jnp.mean(out[:, 1])
    reg = jnp.mean(out[:, 2])
    total = _ALPHA * var + _BETA * dist + _GAMMA * reg
    return (total, var, dist, reg)


# speedup vs baseline: 20.4467x; 1.0473x over previous
import jax
import jax.numpy as jnp
from jax import lax
from jax.experimental import pallas as pl
from jax.experimental.pallas import tpu as pltpu
from jax.experimental.pallas import tpu_sc as plsc

_DELTA_V = 0.5
_DELTA_D = 1.5
_ALPHA = 1.0
_BETA = 1.0
_GAMMA = 0.001
_K = 64

_B = 8
_N = 32768
_D = 16
_L = 16

_WPR = 4
_PTS = _N // _WPR
_CH = 2048
_NCH = _PTS // _CH
_TPC = _CH // _L
_U = 4

_ACC = 1024 + 64 + 16


def _rsqrt_nr(x):
    i = plsc.bitcast(x, jnp.int32)
    i = jnp.int32(0x5F3759DF) - lax.shift_right_logical(i, 1)
    y = plsc.bitcast(i, jnp.float32)
    xh = x * 0.5
    for _ in range(3):
        y = y * (1.5 - xh * y * y)
    return y


def _sqrt16(x):
    return x * _rsqrt_nr(jnp.maximum(x, 1e-30))


def _body(emb_hbm, ids_hbm, out_hbm, xb0, xb1, idsbuf, acc, tmp, mm, mt, pi,
          res, sem0, sem1, sh_acc, sh_mm, sh_pi):
    c = lax.axis_index("c")
    s = lax.axis_index("s")
    row = c * 4 + s // _WPR
    q = s % _WPR
    qbase = q * _PTS
    slot = s // _WPR

    iota = lax.iota(jnp.int32, _L)
    iotad = lax.shift_left(iota, 4)
    ones = jnp.ones((_L,), jnp.float32)

    def _z(i, _):
        acc[pl.ds(i * _L, _L)] = jnp.zeros((_L,), jnp.float32)
        return 0

    lax.fori_loop(0, _ACC // _L, _z, 0)
    for kv in range(4):
        pi[pl.ds(kv * _L, _L)] = jnp.zeros((_L,), jnp.float32)

    pltpu.sync_copy(ids_hbm.at[row, pl.ds(qbase, _PTS)], idsbuf)

    def _emb_copy(ci, xb, sem):
        start = qbase + ci * _CH
        return pltpu.make_async_copy(
            emb_hbm.at[pl.ds(row * (_N * _D) + start * _D, _CH * _D)], xb, sem
        )

    def _buf(ci):
        return (xb0, sem0) if ci % 2 == 0 else (xb1, sem1)

    _emb_copy(0, xb0, sem0).start()
    for ci in range(_NCH):
        xb, sem = _buf(ci)
        if ci + 1 < _NCH:
            nxb, nsem = _buf(ci + 1)
            _emb_copy(ci + 1, nxb, nsem).start()
        _emb_copy(ci, xb, sem).wait()

        @plsc.parallel_loop(0, _TPC, step=1, unroll=_U)
        def _tile1(t, xb=xb, ci=ci):
            ids16 = idsbuf[pl.ds(ci * _CH + t * _L, _L)]
            plsc.addupdate_scatter(acc, [ids16 + 1024], ones)
            off = t * (_L * _D)
            i0 = ids16[0]

            @pl.when(i0 == ids16[_L - 1])
            def _fast(off=off, i0=i0):
                vs = [xb[pl.ds(off + p * _D, _D)] for p in range(_L)]
                while len(vs) > 1:
                    vs = [vs[i] + vs[i + 1] for i in range(0, len(vs), 2)]
                plsc.addupdate(acc.at[pl.ds(i0 * _D, _D)], vs[0])

            @pl.when(i0 != ids16[_L - 1])
            def _slow(off=off, ids16=ids16):
                idsx = lax.shift_left(ids16, 4)
                pix = iotad + off
                for d in range(_D):
                    col = plsc.load_gather(xb, [pix + d])
                    plsc.addupdate_scatter(acc, [idsx + d], col)

    pltpu.sync_copy(acc, sh_acc.at[pl.ds(s * _ACC, _ACC)])
    plsc.subcore_barrier()

    @pl.when(q == 0)
    def _reduce():
        for part in range(1, _WPR):
            pltpu.sync_copy(sh_acc.at[pl.ds((s + part) * _ACC, _ACC)], tmp)

            def _add(i, _):
                acc[pl.ds(i * _L, _L)] = acc[pl.ds(i * _L, _L)] + tmp[
                    pl.ds(i * _L, _L)
                ]
                return 0

            lax.fori_loop(0, _ACC // _L, _add, 0)

        for kv in range(4):
            cnt16 = acc[pl.ds(1024 + kv * _L, _L)]
            acc[pl.ds(1024 + kv * _L, _L)] = 1.0 / jnp.maximum(cnt16, 1.0)

        def _mean(k, _):
            inv = acc[pl.ds(1024 + k, _L)][0]
            mm[pl.ds(k * _L, _L)] = acc[pl.ds(k * _L, _L)] * inv
            return 0

        lax.fori_loop(0, _K, _mean, 0)
        pltpu.sync_copy(mm, sh_mm.at[pl.ds(slot * 1024, 1024)])

    plsc.subcore_barrier()
    pltpu.sync_copy(sh_mm.at[pl.ds(slot * 1024, 1024)], mm)

    _emb_copy(0, xb0, sem0).start()
    for ci in range(_NCH):
        xb, sem = _buf(ci)
        if ci + 1 < _NCH:
            nxb, nsem = _buf(ci + 1)
            _emb_copy(ci + 1, nxb, nsem).start()
        _emb_copy(ci, xb, sem).wait()

        @plsc.parallel_loop(0, _TPC, step=1, unroll=_U)
        def _tile2(t, xb=xb, ci=ci):
            ids16 = idsbuf[pl.ds(ci * _CH + t * _L, _L)]
            pix = iotad + t * (_L * _D)
            i0 = ids16[0]

            def _ssq_from(mcols, pix=pix, xb=xb):
                parts = [jnp.zeros((_L,), jnp.float32) for _ in range(4)]
                for d in range(_D):
                    col = plsc.load_gather(xb, [pix + d])
                    diff = col - mcols[d]
                    parts[d % 4] = parts[d % 4] + diff * diff
                return ((parts[0] + parts[1]) + (parts[2] + parts[3])) + 1e-12

            def _hinge(ssq, ids16=ids16):
                dist = ssq * _rsqrt_nr(ssq)
                hg = jnp.maximum(dist - _DELTA_V, 0.0)
                plsc.addupdate_scatter(pi, [ids16], hg * hg)

            @pl.when(i0 == ids16[_L - 1])
            def _fast(i0=i0, _ssq_from=_ssq_from, _hinge=_hinge):
                mv = mm[pl.ds(i0 * _D, _D)]
                _hinge(_ssq_from([mv[d] for d in range(_D)]))

            @pl.when(i0 != ids16[_L - 1])
            def _slow(ids16=ids16, _ssq_from=_ssq_from, _hinge=_hinge):
                idsx = lax.shift_left(ids16, 4)
                mcols = [
                    plsc.load_gather(mm, [idsx + d]) for d in range(_D)
                ]
                _hinge(_ssq_from(mcols))

    pltpu.sync_copy(pi, sh_pi.at[pl.ds(s * 64, 64)])
    plsc.subcore_barrier()

    @pl.when(q == 0)
    def _finish():
        for part in range(1, _WPR):
            pltpu.sync_copy(sh_pi.at[pl.ds((s + part) * 64, 64)], tmp.at[pl.ds(0, 64)])
            for kv in range(4):
                pi[pl.ds(kv * _L, _L)] = pi[pl.ds(kv * _L, _L)] + tmp[
                    pl.ds(kv * _L, _L)
                ]

        var = jnp.float32(0.0)
        for kv in range(4):
            inv16 = acc[pl.ds(1024 + kv * _L, _L)]
            t16 = pi[pl.ds(kv * _L, _L)] * inv16
            var = var + jnp.sum(t16)
        var = var * jnp.float32(1.0 / _K)

        for d in range(_D):
            for jv in range(4):
                jidx = lax.shift_left(
                    jnp.full((_L,), jv * _L, jnp.int32) + iota, 4
                )
                mt[pl.ds(d * _K + jv * _L, _L)] = plsc.load_gather(
                    mm, [jidx + d]
                )

        dloss = jnp.float32(0.0)
        for jv in range(4):
            mtv = [mt[pl.ds(d * _K + jv * _L, _L)] for d in range(_D)]
            jids = jnp.full((_L,), jv * _L, jnp.int32) + iota

            def _prow(i, dl):
                mi = mm[pl.ds(i * _L, _L)]
                sq = jnp.zeros((_L,), jnp.float32)
                for d in range(_D):
                    diff = mtv[d] - mi[d]
                    sq = sq + diff * diff
                pd = _sqrt16(sq)
                h = jnp.maximum(2.0 * _DELTA_D - pd, 0.0)
                h = jnp.where(jids == i, 0.0, h * h)
                return dl + jnp.sum(h)

            dloss = lax.fori_loop(0, _K, _prow, dloss)
        dloss = dloss * jnp.float32(0.5 / (_K * (_K - 1) / 2.0))

        reg = jnp.float32(0.0)
        for jv in range(4):
            sq = jnp.full((_L,), 1e-12, jnp.float32)
            for d in range(_D):
                v = mt[pl.ds(d * _K + jv * _L, _L)]
                sq = sq + v * v
            reg = reg + jnp.sum(_sqrt16(sq))
        reg = reg * jnp.float32(1.0 / _K)

        out16 = jnp.where(
            iota == 0,
            var,
            jnp.where(iota == 1, dloss, jnp.where(iota == 2, reg, 0.0)),
        )
        res[...] = out16
        pltpu.sync_copy(res, out_hbm.at[row])


def _make_kernel():
    mesh = plsc.VectorSubcoreMesh(core_axis_name="c", subcore_axis_name="s")
    scratch = [
        pltpu.VMEM((_CH * _D,), jnp.float32),
        pltpu.VMEM((_CH * _D,), jnp.float32),
        pltpu.VMEM((_PTS,), jnp.int32),
        pltpu.VMEM((_ACC,), jnp.float32),
        pltpu.VMEM((_ACC,), jnp.float32),
        pltpu.VMEM((1024,), jnp.float32),
        pltpu.VMEM((1024,), jnp.float32),
        pltpu.VMEM((64,), jnp.float32),
        pltpu.VMEM((_L,), jnp.float32),
        pltpu.SemaphoreType.DMA,
        pltpu.SemaphoreType.DMA,
        pltpu.VMEM_SHARED((16 * _ACC,), jnp.float32),
        pltpu.VMEM_SHARED((4 * 1024,), jnp.float32),
        pltpu.VMEM_SHARED((16 * 64,), jnp.float32),
    ]
    return pl.kernel(
        _body,
        mesh=mesh,
        out_type=jax.ShapeDtypeStruct((_B, _L), jnp.float32),
        scratch_types=scratch,
        compiler_params=pltpu.CompilerParams(
            needs_layout_passes=False, use_tc_tiling_on_sc=False
        ),
    )


@jax.jit
def kernel(embeddings, instance_ids):
    out = _make_kernel()(embeddings.reshape(-1), instance_ids)
    var = jnp.mean(out[:, 0])
    dist = jnp.mean(out[:, 1])
    reg = jnp.mean(out[:, 2])
    total = _ALPHA * var + _BETA * dist + _GAMMA * reg
    return (total, var, dist, reg)
